# baseline (speedup 1.0000x reference)
import functools

import jax
import jax.numpy as jnp
from jax import lax
from jax.experimental import pallas as pl
from jax.experimental.pallas import tpu as pltpu
from jax.experimental.pallas import tpu_sc as plsc

VOC = 128
_B, _C, _W = 1024, 26, 20
N = _B * _C * _W
L = 16

_INFO = plsc.get_sparse_core_info()
NC = _INFO.num_cores
NS = _INFO.num_subcores
NW = NC * NS
ROWS_W = N // NW

CHUNK = 256
NCHUNK = ROWS_W // CHUNK
GROUPS = CHUNK // L
B_CH = _B // CHUNK


NBUF = 3
NMAIN = ((NCHUNK - NBUF) // NBUF) * NBUF + NBUF


def _onehot_body(x_hbm, out_hbm, idx_v, buf_a, buf_b, buf_c, sem0, sem1, sem2):
    wid = lax.axis_index("s") * NC + lax.axis_index("c")
    base_row = wid * ROWS_W
    base_chunk = wid * NCHUNK
    bufs = (buf_a, buf_b, buf_c)
    sems = (sem0, sem1, sem2)

    pltpu.sync_copy(x_hbm.at[pl.ds(base_row, ROWS_W)], idx_v)

    iota = lax.iota(jnp.int32, L)
    ones = jnp.full((L,), 1.0, jnp.float32)
    zeros = jnp.zeros((L,), jnp.float32)

    def zinit(buf):
        def body(i, carry):
            for u in range(VOC // L):
                buf[i, pl.ds(u * L, L)] = zeros
            return carry

        lax.fori_loop(0, CHUNK, body, 0)

    def scatter_chunk(buf, ci, val):
        local = ci * CHUNK

        def g_body(g, carry):
            xv = idx_v[pl.ds(local + g * L, L)]
            plsc.store_scatter(buf, [g * L + iota, xv], val)
            return carry

        lax.fori_loop(0, GROUPS, g_body, 0)

    def dma(buf, ci, sem):
        g = base_chunk + ci
        s = g // B_CH
        b0 = (g % B_CH) * CHUNK
        dst = out_hbm.at[s // _W, s % _W, pl.ds(b0, CHUNK)]
        return pltpu.make_async_copy(buf, dst, sem)

    for k in range(NBUF):
        zinit(bufs[k])
        scatter_chunk(bufs[k], k, ones)
        dma(bufs[k], k, sems[k]).start()

    def outer(cc, carry):
        c = cc * NBUF
        for k in range(NBUF):
            ci = c + k
            dma(bufs[k], ci - NBUF, sems[k]).wait()
            scatter_chunk(bufs[k], ci - NBUF, zeros)
            scatter_chunk(bufs[k], ci, ones)
            dma(bufs[k], ci, sems[k]).start()
        return carry

    lax.fori_loop(1, NMAIN // NBUF, outer, 0)

    for ci in range(NMAIN, NCHUNK):
        k = ci % NBUF
        dma(bufs[k], ci - NBUF, sems[k]).wait()
        scatter_chunk(bufs[k], ci - NBUF, zeros)
        scatter_chunk(bufs[k], ci, ones)
        dma(bufs[k], ci, sems[k]).start()

    for ci in range(NCHUNK - NBUF, NCHUNK):
        k = ci % NBUF
        dma(bufs[k], ci, sems[k]).wait()


_onehot = functools.partial(
    pl.kernel,
    mesh=plsc.VectorSubcoreMesh(core_axis_name="c", subcore_axis_name="s"),
    compiler_params=pltpu.CompilerParams(
        needs_layout_passes=False, use_tc_tiling_on_sc=True
    ),
    out_type=jax.ShapeDtypeStruct((_C, _W, _B, VOC), jnp.float32),
    scratch_types=[
        pltpu.VMEM((ROWS_W,), jnp.int32),
        pltpu.VMEM((CHUNK, VOC), jnp.float32),
        pltpu.VMEM((CHUNK, VOC), jnp.float32),
        pltpu.VMEM((CHUNK, VOC), jnp.float32),
        pltpu.SemaphoreType.DMA,
        pltpu.SemaphoreType.DMA,
        pltpu.SemaphoreType.DMA,
    ],
)(_onehot_body)


@jax.jit
def kernel(x):
    xt = jnp.transpose(x, (1, 2, 0)).reshape(N)
    out = _onehot(xt)
    return jnp.transpose(out, (2, 0, 1, 3))


# speedup vs baseline: 1.0275x; 1.0275x over previous
import functools

import jax
import jax.numpy as jnp
from jax import lax
from jax.experimental import pallas as pl
from jax.experimental.pallas import tpu as pltpu
from jax.experimental.pallas import tpu_sc as plsc

VOC = 128
_B, _C, _W = 1024, 26, 20
N = _B * _C * _W
L = 16

_INFO = plsc.get_sparse_core_info()
NC = _INFO.num_cores
NS = _INFO.num_subcores
NW = NC * NS
ROWS_W = N // NW

CHUNK = 256
NCHUNK = ROWS_W // CHUNK
GROUPS = CHUNK // L
B_CH = _B // CHUNK
NSLAB = ROWS_W // _B + 1


def _onehot_body(x_hbm, out_hbm, idx_v, buf_a, buf_b, sem0, sem1, isem):
    wid = lax.axis_index("s") * NC + lax.axis_index("c")
    base_chunk = wid * NCHUNK
    slab0 = base_chunk // B_CH
    off = (base_chunk % B_CH) * CHUNK

    stages = []
    for j in range(NSLAB):
        s = slab0 + j
        cp = pltpu.make_async_copy(
            x_hbm.at[s // _W, s % _W], idx_v.at[pl.ds(j * _B, _B)], isem
        )
        cp.start()
        stages.append(cp)

    iota = lax.iota(jnp.int32, L)
    ones = jnp.full((L,), 1.0, jnp.float32)
    zeros = jnp.zeros((L,), jnp.float32)

    def zinit(buf):
        def body(i, carry):
            for u in range(VOC // L):
                buf[i, pl.ds(u * L, L)] = zeros
            return carry

        lax.fori_loop(0, CHUNK, body, 0)

    def scatter_chunk(buf, ci, val):
        local = off + ci * CHUNK

        def g_body(g, carry):
            xv = idx_v[pl.ds(local + g * L, L)]
            plsc.store_scatter(buf, [g * L + iota, xv], val)
            return carry

        lax.fori_loop(0, GROUPS, g_body, 0)

    def dma(buf, ci, sem):
        g = base_chunk + ci
        s = g // B_CH
        b0 = (g % B_CH) * CHUNK
        dst = out_hbm.at[s // _W, s % _W, pl.ds(b0, CHUNK)]
        return pltpu.make_async_copy(buf, dst, sem)

    zinit(buf_a)
    for cp in stages:
        cp.wait()

    scatter_chunk(buf_a, 0, ones)
    dma(buf_a, 0, sem0).start()
    zinit(buf_b)
    scatter_chunk(buf_b, 1, ones)
    dma(buf_b, 1, sem1).start()

    def outer(cc, carry):
        c = cc * 2
        for buf, b_, sem in ((buf_a, 0, sem0), (buf_b, 1, sem1)):
            ci = c + b_
            dma(buf, ci - 2, sem).wait()
            scatter_chunk(buf, ci - 2, zeros)
            scatter_chunk(buf, ci, ones)
            dma(buf, ci, sem).start()
        return carry

    lax.fori_loop(1, (NCHUNK - 1) // 2, outer, 0)

    dma(buf_a, NCHUNK - 3, sem0).wait()
    scatter_chunk(buf_a, NCHUNK - 3, zeros)
    scatter_chunk(buf_a, NCHUNK - 1, ones)
    dma(buf_a, NCHUNK - 1, sem0).start()

    dma(buf_b, NCHUNK - 2, sem1).wait()
    dma(buf_a, NCHUNK - 1, sem0).wait()


_onehot = functools.partial(
    pl.kernel,
    mesh=plsc.VectorSubcoreMesh(core_axis_name="c", subcore_axis_name="s"),
    compiler_params=pltpu.CompilerParams(
        needs_layout_passes=False, use_tc_tiling_on_sc=True
    ),
    out_type=jax.ShapeDtypeStruct((_C, _W, _B, VOC), jnp.float32),
    scratch_types=[
        pltpu.VMEM((NSLAB * _B,), jnp.int32),
        pltpu.VMEM((CHUNK, VOC), jnp.float32),
        pltpu.VMEM((CHUNK, VOC), jnp.float32),
        pltpu.SemaphoreType.DMA,
        pltpu.SemaphoreType.DMA,
        pltpu.SemaphoreType.DMA,
    ],
)(_onehot_body)


@jax.jit
def kernel(x):
    xt = jnp.transpose(x, (1, 2, 0))
    out = _onehot(xt)
    return jnp.transpose(out, (2, 0, 1, 3))
